# baseline (speedup 1.0000x reference)
import functools

import jax
import jax.numpy as jnp
from jax import lax
from jax.experimental import pallas as pl
from jax.experimental.pallas import tpu as pltpu
from jax.experimental.pallas import tpu_sc as plsc

F = 26
CARD = 1000
D = 128
B = 4096
L = 16
NC, NS = 2, 16
NW = NC * NS
ROWS = B * F
RPW = ROWS // NW
CHUNK = 128
NCH = RPW // CHUNK
VPR = D // L
NB = 5

_mesh = plsc.VectorSubcoreMesh(core_axis_name="c", subcore_axis_name="s")


@functools.partial(
    pl.kernel,
    out_type=jax.ShapeDtypeStruct((ROWS, D), jnp.float32),
    mesh=_mesh,
    scratch_types=[
        pltpu.VMEM((RPW,), jnp.int32),
        pltpu.VMEM((F, D), jnp.float32),
    ]
    + [pltpu.VMEM((CHUNK, D), jnp.float32) for _ in range(NB)]
    + [pltpu.SemaphoreType.DMA for _ in range(2 * NB)],
)
def _tokenize(idx_hbm, tab_hbm, bias_hbm, out_hbm, gid_v, bias_v, *bufs_sems):
    bufq = bufs_sems[:NB]
    sem_g = bufs_sems[NB:2 * NB]
    sem_w = bufs_sems[2 * NB:]
    wid = lax.axis_index("s") * NC + lax.axis_index("c")
    wbase = wid * RPW
    lane = lax.iota(jnp.int32, L)

    pltpu.sync_copy(idx_hbm.at[pl.ds(wbase, RPW)], gid_v)
    pltpu.sync_copy(bias_hbm, bias_v)
    for g in range(RPW // L):
        sl = pl.ds(g * L, L)
        fvec = lax.shift_right_logical(wbase + g * L + lane, 12)
        gid_v[sl] = jnp.maximum(gid_v[sl], 0) + fvec * CARD

    gd, wd = {}, {}

    def fire(k):
        gd[k] = pltpu.async_copy(
            tab_hbm.at[gid_v.at[pl.ds(k * CHUNK, CHUNK)]], bufq[k % NB], sem_g[k % NB]
        )

    fire(0)
    fire(1)
    fire(2)
    for k in range(NCH):
        s = k % NB
        if k + 3 < NCH:
            if k - 2 >= 0:
                wd[k - 2].wait()
            fire(k + 3)
        gd[k].wait()
        buf = bufq[s]
        fk = lax.shift_right_logical(wbase + k * CHUNK, 12)
        bvals = [bias_v[fk, pl.ds(j * L, L)] for j in range(VPR)]

        @pl.loop(0, CHUNK, unroll=2)
        def _bias_add(r):
            for j in range(VPR):
                sl = pl.ds(j * L, L)
                buf[r, sl] = buf[r, sl] + bvals[j]

        wd[k] = pltpu.async_copy(
            buf, out_hbm.at[pl.ds(wbase + k * CHUNK, CHUNK), :], sem_w[s]
        )

    for k in range(max(0, NCH - 5), NCH):
        wd[k].wait()


def kernel(x_cat, tables, bias):
    idx_fmajor = x_cat.astype(jnp.int32).T.reshape(ROWS)
    tab = tables.reshape(F * CARD, D)
    out = _tokenize(idx_fmajor, tab, bias)
    return out.reshape(F, B, D).transpose(1, 0, 2)
